# Initial kernel scaffold; baseline (speedup 1.0000x reference)
import jax
import jax.numpy as jnp
from jax.experimental import pallas as pl


def kernel(x, edge_index, W1l, b1l, W1r, g1, be1, W2l, b2l, W2r, g2, be2):
    raise NotImplementedError("write your pallas kernel here")


# speedup vs baseline: 8.3508x; 8.3508x over previous
import functools

import jax
import jax.numpy as jnp
from jax import lax
from jax.experimental import pallas as pl
from jax.experimental.pallas import tpu as pltpu
from jax.experimental.pallas import tpu_sc as plsc

N = 100000
E = 1600000
HID = 64
EPS = 1e-5
CW = 16
NCHUNK = HID // CW

NC, NS = 2, 16
NW = NC * NS
EPW = E // NW
EB = 1000
NEB = EPW // EB
RPS = N // NS

BN = 2000
NBN = N // BN


def _sc_seg_sum_body(nk, src_hbm, dst_hbm, zero_hbm, *rest):
    tabs = rest[:nk]
    outs = rest[nk:2 * nk]
    idx_s, idx_d, rows, sem, acc = rest[2 * nk:]
    cid = lax.axis_index("c")
    sid = lax.axis_index("s")
    wid = cid * NS + sid
    ebase = wid * EPW
    r0 = sid * RPS
    for k in range(nk):
        pltpu.sync_copy(zero_hbm, acc.at[pl.ds(r0, RPS)])
        plsc.subcore_barrier()

        def blk(b, carry):
            e0 = ebase + b * EB
            pltpu.sync_copy(src_hbm.at[pl.ds(e0, EB)], idx_s)
            pltpu.sync_copy(dst_hbm.at[pl.ds(e0, EB)], idx_d)
            pltpu.async_copy(tabs[k].at[idx_s], rows, sem).wait()
            pltpu.sync_copy(rows, acc.at[idx_d], add=True)
            return carry

        lax.fori_loop(0, NEB, blk, 0)
        plsc.subcore_barrier()
        pltpu.sync_copy(acc.at[pl.ds(r0, RPS)],
                        outs[k].at[pl.ds(cid * N + r0, RPS)])
        plsc.subcore_barrier()


def _sc_seg_sum(src, dst, zero_blk, tables):
    nk = len(tables)
    mesh = plsc.VectorSubcoreMesh(core_axis_name="c", subcore_axis_name="s",
                                  num_cores=NC, num_subcores=NS)
    fn = pl.kernel(
        functools.partial(_sc_seg_sum_body, nk),
        out_type=[jax.ShapeDtypeStruct((2 * N, CW), jnp.float32)] * nk,
        mesh=mesh,
        compiler_params=pltpu.CompilerParams(use_tc_tiling_on_sc=False),
        scratch_types=[
            pltpu.VMEM((EB,), jnp.int32),
            pltpu.VMEM((EB,), jnp.int32),
            pltpu.VMEM((EB, CW), jnp.float32),
            pltpu.SemaphoreType.DMA,
            pltpu.VMEM_SHARED((N, CW), jnp.float32),
        ],
    )
    return fn(src, dst, zero_blk, *tables)


def _l1_h(aggA, aggB, xpad, Wl, Wr, b):
    agg = aggA[...] + aggB[...]
    deg = jnp.maximum(agg[:, 2:3], 1.0)
    dinv = 1.0 / deg
    meanp = agg * dinv
    h = (jnp.dot(meanp, Wl[...], preferred_element_type=jnp.float32)
         + jnp.dot(xpad[...], Wr[...], preferred_element_type=jnp.float32)
         + b[...])
    return h, dinv


def _stats_update(j, h, g, be, ac_out, s_sum, s_sq):
    @pl.when(j == 0)
    def _():
        s_sum[...] = jnp.zeros_like(s_sum)
        s_sq[...] = jnp.zeros_like(s_sq)

    s_sum[...] += jnp.sum(h, axis=0, keepdims=True)
    s_sq[...] += jnp.sum(h * h, axis=0, keepdims=True)

    @pl.when(j == NBN - 1)
    def _():
        mu = s_sum[...] / N
        var = s_sq[...] / N - mu * mu
        a = g[...] * lax.rsqrt(var + EPS)
        ac_out[0:1, :] = a
        ac_out[1:2, :] = be[...] - mu * a


def _l1_stats_body(aggA, aggB, xpad, Wl, Wr, b, g, be, ac_out, s_sum, s_sq):
    h, _ = _l1_h(aggA, aggB, xpad, Wl, Wr, b)
    _stats_update(pl.program_id(0), h, g, be, ac_out, s_sum, s_sq)


def _l1_apply_body(aggA, aggB, xpad, Wl, Wr, b, ac,
                   out0, out1, out2, out3, dinv_out):
    h, dinv = _l1_h(aggA, aggB, xpad, Wl, Wr, b)
    y = jnp.maximum(h * ac[0:1, :] + ac[1:2, :], 0.0)
    out0[...] = y[:, 0:16]
    out1[...] = y[:, 16:32]
    out2[...] = y[:, 32:48]
    out3[...] = y[:, 48:64]
    dinv_out[...] = jnp.broadcast_to(dinv, (BN, CW))


def _blkspecs():
    blk = pl.BlockSpec((BN, CW), lambda j: (j, 0))
    blkB = pl.BlockSpec((BN, CW), lambda j: (NBN + j, 0))
    wspec = lambda r, c: pl.BlockSpec((r, c), lambda j: (0, 0))
    return blk, blkB, wspec


def _tc_layer1(agg1, xpad, Wl, Wr, b, g, be):
    blk, blkB, wspec = _blkspecs()
    ac = pl.pallas_call(
        _l1_stats_body,
        grid=(NBN,),
        in_specs=[blk, blkB, blk, wspec(16, 64), wspec(16, 64),
                  wspec(1, 64), wspec(1, 64), wspec(1, 64)],
        out_specs=wspec(2, 64),
        out_shape=jax.ShapeDtypeStruct((2, 64), jnp.float32),
        scratch_shapes=[pltpu.VMEM((1, 64), jnp.float32),
                        pltpu.VMEM((1, 64), jnp.float32)],
    )(agg1, agg1, xpad, Wl, Wr, b, g, be)
    outs = pl.pallas_call(
        _l1_apply_body,
        grid=(NBN,),
        in_specs=[blk, blkB, blk, wspec(16, 64), wspec(16, 64),
                  wspec(1, 64), wspec(2, 64)],
        out_specs=[blk] * 5,
        out_shape=[jax.ShapeDtypeStruct((N, CW), jnp.float32)] * 5,
    )(agg1, agg1, xpad, Wl, Wr, b, ac)
    return outs


def _l2_h(refs):
    (a0A, a0B, a1A, a1B, a2A, a2B, a3A, a3B,
     h0, h1c, h2c, h3, dinv_in, Wl, Wr, b) = refs
    agg = jnp.concatenate([a0A[...] + a0B[...], a1A[...] + a1B[...],
                           a2A[...] + a2B[...], a3A[...] + a3B[...]], axis=1)
    h1r = jnp.concatenate([h0[...], h1c[...], h2c[...], h3[...]], axis=1)
    mean2 = agg * dinv_in[:, 0:1]
    return (jnp.dot(mean2, Wl[...], preferred_element_type=jnp.float32)
            + jnp.dot(h1r, Wr[...], preferred_element_type=jnp.float32)
            + b[...])


def _l2_stats_body(*refs):
    ins, g, be, ac_out, s_sum, s_sq = refs[:16], refs[16], refs[17], refs[18], refs[19], refs[20]
    h = _l2_h(ins)
    _stats_update(pl.program_id(0), h, g, be, ac_out, s_sum, s_sq)


def _l2_apply_body(*refs):
    ins, ac, out, s_max = refs[:16], refs[16], refs[17], refs[18]
    j = pl.program_id(0)
    h = _l2_h(ins)
    y = jnp.maximum(h * ac[0:1, :] + ac[1:2, :], 0.0)

    @pl.when(j == 0)
    def _():
        s_max[...] = jnp.zeros_like(s_max)

    s_max[...] = jnp.maximum(s_max[...], jnp.max(y, axis=0, keepdims=True))

    @pl.when(j == NBN - 1)
    def _():
        out[...] = s_max[...]


def _tc_layer2(agg2s, h1cs, dinv, Wl, Wr, b, g, be):
    blk, blkB, wspec = _blkspecs()
    args = []
    specs = []
    for a in agg2s:
        args += [a, a]
        specs += [blk, blkB]
    args += list(h1cs) + [dinv, Wl, Wr, b]
    specs += [blk] * 5 + [wspec(64, 64), wspec(64, 64), wspec(1, 64)]
    ac = pl.pallas_call(
        _l2_stats_body,
        grid=(NBN,),
        in_specs=specs + [wspec(1, 64), wspec(1, 64)],
        out_specs=wspec(2, 64),
        out_shape=jax.ShapeDtypeStruct((2, 64), jnp.float32),
        scratch_shapes=[pltpu.VMEM((1, 64), jnp.float32),
                        pltpu.VMEM((1, 64), jnp.float32)],
    )(*args, g, be)
    return pl.pallas_call(
        _l2_apply_body,
        grid=(NBN,),
        in_specs=specs + [wspec(2, 64)],
        out_specs=wspec(1, 64),
        out_shape=jax.ShapeDtypeStruct((1, 64), jnp.float32),
        scratch_shapes=[pltpu.VMEM((1, 64), jnp.float32)],
    )(*args, ac)


def kernel(x, edge_index, W1l, b1l, W1r, g1, be1, W2l, b2l, W2r, g2, be2):
    f32 = jnp.float32
    src = edge_index[0]
    dst = edge_index[1]
    xpad = jnp.concatenate(
        [x, jnp.ones((N, 1), f32), jnp.zeros((N, CW - 3), f32)], axis=1)
    zero_blk = jnp.zeros((RPS, CW), f32)

    W1l_p = jnp.zeros((CW, HID), f32).at[0:2].set(W1l)
    W1r_p = jnp.zeros((CW, HID), f32).at[0:2].set(W1r)

    (agg1,) = _sc_seg_sum(src, dst, zero_blk, [xpad])
    h1c0, h1c1, h1c2, h1c3, dinv = _tc_layer1(
        agg1, xpad, W1l_p, W1r_p,
        b1l.reshape(1, HID), g1.reshape(1, HID), be1.reshape(1, HID))
    agg2s = _sc_seg_sum(src, dst, zero_blk, [h1c0, h1c1, h1c2, h1c3])
    out = _tc_layer2(
        agg2s, (h1c0, h1c1, h1c2, h1c3), dinv, W2l, W2r,
        b2l.reshape(1, HID), g2.reshape(1, HID), be2.reshape(1, HID))
    return out.reshape(HID)


# speedup vs baseline: 16.4549x; 1.9705x over previous
import functools

import jax
import jax.numpy as jnp
from jax import lax
from jax.experimental import pallas as pl
from jax.experimental.pallas import tpu as pltpu
from jax.experimental.pallas import tpu_sc as plsc

N = 100000
NPAD = 102400
E = 1600000
HID = 64
EPS = 1e-5
CW = 16
NCHUNK = HID // CW

NC, NS = 2, 16
NW = NC * NS
EPW = E // NW
EB = 400
NEB = EPW // EB
RPS = NPAD // NS


def _sc_seg_sum_body(nk, ei_hbm, zero_hbm, *rest):
    tabs = rest[:nk]
    outs = rest[nk:2 * nk]
    idxb, rows, sem_i, sem_g, sem_s, acc = rest[2 * nk:]
    cid = lax.axis_index("c")
    sid = lax.axis_index("s")
    wid = cid * NS + sid
    ebase = wid * EPW
    r0 = sid * RPS
    for k in range(nk):
        tab = tabs[k]
        pltpu.sync_copy(zero_hbm, acc.at[pl.ds(r0, RPS)])
        plsc.subcore_barrier()

        pltpu.sync_copy(ei_hbm.at[:, pl.ds(ebase, EB)], idxb.at[0])
        pltpu.async_copy(tab.at[idxb.at[0, 0]], rows.at[0], sem_g)
        pltpu.async_copy(ei_hbm.at[:, pl.ds(ebase + EB, EB)],
                         idxb.at[1], sem_i)

        def blk(i, carry):
            p3 = lax.rem(i, 3)
            p4 = lax.rem(i, 4)
            n3 = lax.rem(i + 1, 3)
            n4 = lax.rem(i + 1, 4)

            @pl.when(i >= 2)
            def _():
                pltpu.make_async_copy(
                    rows.at[n3], acc.at[pl.ds(0, EB)], sem_s).wait()

            @pl.when(i + 1 < NEB)
            def _():
                pltpu.make_async_copy(
                    ei_hbm.at[:, pl.ds(ebase, EB)], idxb.at[n4], sem_i).wait()
                pltpu.async_copy(tab.at[idxb.at[n4, 0]], rows.at[n3], sem_g)

            @pl.when(i + 2 < NEB)
            def _():
                pltpu.async_copy(
                    ei_hbm.at[:, pl.ds(ebase + (i + 2) * EB, EB)],
                    idxb.at[lax.rem(i + 2, 4)], sem_i)

            pltpu.make_async_copy(
                tab.at[pl.ds(0, EB)], rows.at[p3], sem_g).wait()
            pltpu.async_copy(rows.at[p3], acc.at[idxb.at[p4, 1]], sem_s,
                             add=True)
            return carry

        lax.fori_loop(0, NEB, blk, 0)
        pltpu.make_async_copy(
            rows.at[lax.rem(NEB - 2, 3)], acc.at[pl.ds(0, EB)], sem_s).wait()
        pltpu.make_async_copy(
            rows.at[lax.rem(NEB - 1, 3)], acc.at[pl.ds(0, EB)], sem_s).wait()
        plsc.subcore_barrier()
        pltpu.sync_copy(acc.at[pl.ds(r0, RPS)],
                        outs[k].at[pl.ds(cid * NPAD + r0, RPS)])
        plsc.subcore_barrier()


def _sc_seg_sum(edge_index, zero_blk, tables):
    nk = len(tables)
    mesh = plsc.VectorSubcoreMesh(core_axis_name="c", subcore_axis_name="s",
                                  num_cores=NC, num_subcores=NS)
    fn = pl.kernel(
        functools.partial(_sc_seg_sum_body, nk),
        out_type=[jax.ShapeDtypeStruct((2 * NPAD, CW), jnp.float32)] * nk,
        mesh=mesh,
        compiler_params=pltpu.CompilerParams(use_tc_tiling_on_sc=False),
        scratch_types=[
            pltpu.VMEM((4, 2, EB), jnp.int32),
            pltpu.VMEM((3, EB, CW), jnp.float32),
            pltpu.SemaphoreType.DMA,
            pltpu.SemaphoreType.DMA,
            pltpu.SemaphoreType.DMA,
            pltpu.VMEM_SHARED((NPAD, CW), jnp.float32),
        ],
    )
    return fn(edge_index, zero_blk, *tables)


NP = NPAD // 8
NPB = N // 8
BNP = 1600
NBP = NP // BNP


def _fold8(v, op):
    acc = v[:, 0:64]
    for s in range(1, 8):
        acc = op(acc, v[:, s * 64:(s + 1) * 64])
    return acc


def _tile8(v):
    return jnp.concatenate([v] * 8, axis=1)


def _dinv_packed(agg):
    parts = []
    for s in range(8):
        d = jnp.maximum(agg[:, s * 16 + 2:s * 16 + 3], 1.0)
        parts.append(jnp.broadcast_to(1.0 / d, (BNP, CW)))
    return jnp.concatenate(parts, axis=1)


def _to512(chunks, width):
    parts = []
    for s in range(8):
        for c in chunks:
            parts.append(c[:, s * width:(s + 1) * width])
    return jnp.concatenate(parts, axis=1)


def _valid_mask(j):
    rows = j * BNP + lax.broadcasted_iota(jnp.int32, (BNP, 512), 0)
    return rows < NPB


def _stats_update(j, h, g, be, ac_out, s_sum, s_sq):
    @pl.when(j == 0)
    def _():
        s_sum[...] = jnp.zeros_like(s_sum)
        s_sq[...] = jnp.zeros_like(s_sq)

    h = jnp.where(_valid_mask(j), h, 0.0)
    s_sum[...] += jnp.sum(h, axis=0, keepdims=True)
    s_sq[...] += jnp.sum(h * h, axis=0, keepdims=True)

    @pl.when(j == NBP - 1)
    def _():
        add = lambda p, q: p + q
        mu = _fold8(s_sum[...], add) / N
        var = _fold8(s_sq[...], add) / N - mu * mu
        a = g[...] * lax.rsqrt(var + EPS)
        ac_out[0:1, :] = a
        ac_out[1:2, :] = be[...] - mu * a


def _l1_h(aggA, aggB, xpadP, Wl, Wr, b):
    agg = aggA[...] + aggB[...]
    dinvP = _dinv_packed(agg)
    meanp = agg * dinvP
    h = (jnp.dot(meanp, Wl[...], preferred_element_type=jnp.float32)
         + jnp.dot(xpadP[...], Wr[...], preferred_element_type=jnp.float32)
         + b[...])
    return h, dinvP


def _l1_stats_body(aggA, aggB, xpadP, Wl, Wr, b, g, be, ac_out, s_sum, s_sq):
    h, _ = _l1_h(aggA, aggB, xpadP, Wl, Wr, b)
    _stats_update(pl.program_id(0), h, g, be, ac_out, s_sum, s_sq)


def _l1_apply_body(aggA, aggB, xpadP, Wl, Wr, b, ac,
                   out0, out1, out2, out3, dinv_out):
    h, dinvP = _l1_h(aggA, aggB, xpadP, Wl, Wr, b)
    y = jnp.maximum(h * _tile8(ac[0:1, :]) + _tile8(ac[1:2, :]), 0.0)
    outs = (out0, out1, out2, out3)
    for t in range(NCHUNK):
        outs[t][...] = jnp.concatenate(
            [y[:, s * 64 + t * CW:s * 64 + t * CW + CW] for s in range(8)],
            axis=1)
    dinv_out[...] = dinvP


def _blkspecs():
    blk = pl.BlockSpec((BNP, 128), lambda j: (j, 0))
    blkB = pl.BlockSpec((BNP, 128), lambda j: (NBP + j, 0))
    wspec = lambda r, c: pl.BlockSpec((r, c), lambda j: (0, 0))
    return blk, blkB, wspec


def _tc_layer1(agg1P, xpadP, Wl, Wr, b, g, be):
    blk, blkB, wspec = _blkspecs()
    ac = pl.pallas_call(
        _l1_stats_body,
        grid=(NBP,),
        in_specs=[blk, blkB, blk, wspec(128, 512), wspec(128, 512),
                  wspec(1, 512), wspec(1, 64), wspec(1, 64)],
        out_specs=wspec(2, 64),
        out_shape=jax.ShapeDtypeStruct((2, 64), jnp.float32),
        scratch_shapes=[pltpu.VMEM((1, 512), jnp.float32),
                        pltpu.VMEM((1, 512), jnp.float32)],
    )(agg1P, agg1P, xpadP, Wl, Wr, b, g, be)
    outs = pl.pallas_call(
        _l1_apply_body,
        grid=(NBP,),
        in_specs=[blk, blkB, blk, wspec(128, 512), wspec(128, 512),
                  wspec(1, 512), wspec(2, 64)],
        out_specs=[blk] * 5,
        out_shape=[jax.ShapeDtypeStruct((NP, 128), jnp.float32)] * 5,
    )(agg1P, agg1P, xpadP, Wl, Wr, b, ac)
    return outs


def _l2_h(refs):
    (a0A, a0B, a1A, a1B, a2A, a2B, a3A, a3B,
     h0, h1c, h2c, h3, dinv_in, Wl, Wr, b) = refs
    aggc = [a0A[...] + a0B[...], a1A[...] + a1B[...],
            a2A[...] + a2B[...], a3A[...] + a3B[...]]
    agg = _to512(aggc, CW)
    h1r = _to512([h0[...], h1c[...], h2c[...], h3[...]], CW)
    dinvP = dinv_in[...]
    dinv512 = jnp.concatenate(
        [jnp.broadcast_to(dinvP[:, s * 16:s * 16 + 1], (BNP, 64))
         for s in range(8)], axis=1)
    mean2 = agg * dinv512
    return (jnp.dot(mean2, Wl[...], preferred_element_type=jnp.float32)
            + jnp.dot(h1r, Wr[...], preferred_element_type=jnp.float32)
            + b[...])


def _l2_stats_body(*refs):
    ins, g, be, ac_out, s_sum, s_sq = (refs[:16], refs[16], refs[17],
                                       refs[18], refs[19], refs[20])
    h = _l2_h(ins)
    _stats_update(pl.program_id(0), h, g, be, ac_out, s_sum, s_sq)


def _l2_apply_body(*refs):
    ins, ac, out, s_max = refs[:16], refs[16], refs[17], refs[18]
    j = pl.program_id(0)
    h = _l2_h(ins)
    y = jnp.maximum(h * _tile8(ac[0:1, :]) + _tile8(ac[1:2, :]), 0.0)
    y = jnp.where(_valid_mask(j), y, 0.0)

    @pl.when(j == 0)
    def _():
        s_max[...] = jnp.zeros_like(s_max)

    s_max[...] = jnp.maximum(s_max[...], jnp.max(y, axis=0, keepdims=True))

    @pl.when(j == NBP - 1)
    def _():
        out[...] = _fold8(s_max[...], jnp.maximum)


def _tc_layer2(agg2Ps, h1cPs, dinvP, Wl, Wr, b, g, be):
    blk, blkB, wspec = _blkspecs()
    args = []
    specs = []
    for a in agg2Ps:
        args += [a, a]
        specs += [blk, blkB]
    args += list(h1cPs) + [dinvP, Wl, Wr, b]
    specs += [blk] * 5 + [wspec(512, 512), wspec(512, 512), wspec(1, 512)]
    ac = pl.pallas_call(
        _l2_stats_body,
        grid=(NBP,),
        in_specs=specs + [wspec(1, 64), wspec(1, 64)],
        out_specs=wspec(2, 64),
        out_shape=jax.ShapeDtypeStruct((2, 64), jnp.float32),
        scratch_shapes=[pltpu.VMEM((1, 512), jnp.float32),
                        pltpu.VMEM((1, 512), jnp.float32)],
    )(*args, g, be)
    return pl.pallas_call(
        _l2_apply_body,
        grid=(NBP,),
        in_specs=specs + [wspec(2, 64)],
        out_specs=wspec(1, 64),
        out_shape=jax.ShapeDtypeStruct((1, 64), jnp.float32),
        scratch_shapes=[pltpu.VMEM((1, 512), jnp.float32)],
    )(*args, ac)


def kernel(x, edge_index, W1l, b1l, W1r, g1, be1, W2l, b2l, W2r, g2, be2):
    f32 = jnp.float32
    xpad = jnp.concatenate(
        [x, jnp.ones((N, 1), f32), jnp.zeros((N, CW - 3), f32)], axis=1)
    xpad = jnp.pad(xpad, ((0, NPAD - N), (0, 0)))
    zero_blk = jnp.zeros((RPS, CW), f32)
    eye8 = jnp.eye(8, dtype=f32)

    W1l_p = jnp.zeros((CW, HID), f32).at[0:2].set(W1l)
    W1r_p = jnp.zeros((CW, HID), f32).at[0:2].set(W1r)
    W1l_big = jnp.kron(eye8, W1l_p)
    W1r_big = jnp.kron(eye8, W1r_p)
    W2l_big = jnp.kron(eye8, W2l)
    W2r_big = jnp.kron(eye8, W2r)
    b1t = jnp.tile(b1l.reshape(1, HID), (1, 8))
    b2t = jnp.tile(b2l.reshape(1, HID), (1, 8))

    (agg1,) = _sc_seg_sum(edge_index, zero_blk, [xpad])
    h1c0, h1c1, h1c2, h1c3, dinvP = _tc_layer1(
        agg1.reshape(2 * NP, 128), xpad.reshape(NP, 128),
        W1l_big, W1r_big, b1t, g1.reshape(1, HID), be1.reshape(1, HID))
    h1cPs = (h1c0, h1c1, h1c2, h1c3)
    agg2s = _sc_seg_sum(edge_index, zero_blk,
                        [c.reshape(NPAD, CW) for c in h1cPs])
    out = _tc_layer2(
        [a.reshape(2 * NP, 128) for a in agg2s], h1cPs, dinvP,
        W2l_big, W2r_big, b2t, g2.reshape(1, HID), be2.reshape(1, HID))
    return out.reshape(HID)


# speedup vs baseline: 17.0633x; 1.0370x over previous
import functools

import jax
import jax.numpy as jnp
from jax import lax
from jax.experimental import pallas as pl
from jax.experimental.pallas import tpu as pltpu
from jax.experimental.pallas import tpu_sc as plsc

N = 100000
NPAD = 102400
E = 1600000
HID = 64
EPS = 1e-5
CW = 16
NCHUNK = HID // CW

NC, NS = 2, 16
NW = NC * NS
EPW = E // NW
EB = 400
NEB = EPW // EB
RPS = NPAD // NS


def _edge_pipeline(tab, acc, ei_hbm, idxb, rows, sem_i, sem_g, sem_s,
                   ebase, neb):
    pltpu.sync_copy(ei_hbm.at[:, pl.ds(ebase, EB)], idxb.at[0])
    pltpu.async_copy(tab.at[idxb.at[0, 0]], rows.at[0], sem_g)
    pltpu.async_copy(ei_hbm.at[:, pl.ds(ebase + EB, EB)], idxb.at[1], sem_i)

    def blk(i, carry):
        p3 = lax.rem(i, 3)
        p4 = lax.rem(i, 4)
        n3 = lax.rem(i + 1, 3)
        n4 = lax.rem(i + 1, 4)

        @pl.when(i >= 2)
        def _():
            pltpu.make_async_copy(
                rows.at[n3], acc.at[pl.ds(0, EB)], sem_s).wait()

        @pl.when(i + 1 < neb)
        def _():
            pltpu.make_async_copy(
                ei_hbm.at[:, pl.ds(ebase, EB)], idxb.at[n4], sem_i).wait()
            pltpu.async_copy(tab.at[idxb.at[n4, 0]], rows.at[n3], sem_g)

        @pl.when(i + 2 < neb)
        def _():
            pltpu.async_copy(
                ei_hbm.at[:, pl.ds(ebase + (i + 2) * EB, EB)],
                idxb.at[lax.rem(i + 2, 4)], sem_i)

        pltpu.make_async_copy(
            tab.at[pl.ds(0, EB)], rows.at[p3], sem_g).wait()
        pltpu.async_copy(rows.at[p3], acc.at[idxb.at[p4, 1]], sem_s,
                         add=True)
        return carry

    lax.fori_loop(0, neb, blk, 0)
    pltpu.make_async_copy(
        rows.at[lax.rem(neb - 2, 3)], acc.at[pl.ds(0, EB)], sem_s).wait()
    pltpu.make_async_copy(
        rows.at[lax.rem(neb - 1, 3)], acc.at[pl.ds(0, EB)], sem_s).wait()


def _sc_l1_body(ei_hbm, zero_hbm, tab, out,
                idxb, rows, sem_i, sem_g, sem_s, acc):
    cid = lax.axis_index("c")
    sid = lax.axis_index("s")
    r0 = sid * RPS
    pltpu.sync_copy(zero_hbm, acc.at[pl.ds(r0, RPS)])
    plsc.subcore_barrier()
    _edge_pipeline(tab, acc, ei_hbm, idxb, rows, sem_i, sem_g, sem_s,
                   (cid * NS + sid) * EPW, NEB)
    plsc.subcore_barrier()
    pltpu.sync_copy(acc.at[pl.ds(r0, RPS)],
                    out.at[pl.ds(cid * NPAD + r0, RPS)])
    plsc.subcore_barrier()


EPS2 = E // NS
NEB2 = EPS2 // EB


def _sc_l2_body(ei_hbm, zero_hbm, *rest):
    tabs = rest[:4]
    outs = rest[4:8]
    idxb, rows, sem_i, sem_g, sem_s, acc = rest[8:]
    cid = lax.axis_index("c")
    sid = lax.axis_index("s")
    r0 = sid * RPS
    for cv in range(NC):
        @pl.when(cid == cv)
        def _():
            for k in range(2):
                kk = 2 * cv + k
                pltpu.sync_copy(zero_hbm, acc.at[pl.ds(r0, RPS)])
                plsc.subcore_barrier()
                _edge_pipeline(tabs[kk], acc, ei_hbm, idxb, rows,
                               sem_i, sem_g, sem_s, sid * EPS2, NEB2)
                plsc.subcore_barrier()
                pltpu.sync_copy(acc.at[pl.ds(r0, RPS)],
                                outs[kk].at[pl.ds(r0, RPS)])
                plsc.subcore_barrier()


def _sc_mesh():
    return plsc.VectorSubcoreMesh(core_axis_name="c", subcore_axis_name="s",
                                  num_cores=NC, num_subcores=NS)


def _sc_scratch():
    return [
        pltpu.VMEM((4, 2, EB), jnp.int32),
        pltpu.VMEM((3, EB, CW), jnp.float32),
        pltpu.SemaphoreType.DMA,
        pltpu.SemaphoreType.DMA,
        pltpu.SemaphoreType.DMA,
        pltpu.VMEM_SHARED((NPAD, CW), jnp.float32),
    ]


def _sc_seg_sum_l1(edge_index, zero_blk, table):
    fn = pl.kernel(
        _sc_l1_body,
        out_type=jax.ShapeDtypeStruct((2 * NPAD, CW), jnp.float32),
        mesh=_sc_mesh(),
        compiler_params=pltpu.CompilerParams(use_tc_tiling_on_sc=False),
        scratch_types=_sc_scratch(),
    )
    return fn(edge_index, zero_blk, table)


def _sc_seg_sum_l2(edge_index, zero_blk, tables):
    fn = pl.kernel(
        _sc_l2_body,
        out_type=[jax.ShapeDtypeStruct((NPAD, CW), jnp.float32)] * 4,
        mesh=_sc_mesh(),
        compiler_params=pltpu.CompilerParams(use_tc_tiling_on_sc=False),
        scratch_types=_sc_scratch(),
    )
    return fn(edge_index, zero_blk, *tables)


NP = NPAD // 8
NPB = N // 8
BNP = 1600
NBP = NP // BNP


def _fold8(v, op):
    acc = v[:, 0:64]
    for s in range(1, 8):
        acc = op(acc, v[:, s * 64:(s + 1) * 64])
    return acc


def _tile8(v):
    return jnp.concatenate([v] * 8, axis=1)


def _dinv_packed(agg):
    parts = []
    for s in range(8):
        d = jnp.maximum(agg[:, s * 16 + 2:s * 16 + 3], 1.0)
        parts.append(jnp.broadcast_to(1.0 / d, (BNP, CW)))
    return jnp.concatenate(parts, axis=1)


def _to512(chunks, width):
    parts = []
    for s in range(8):
        for c in chunks:
            parts.append(c[:, s * width:(s + 1) * width])
    return jnp.concatenate(parts, axis=1)


def _valid_mask(j):
    rows = j * BNP + lax.broadcasted_iota(jnp.int32, (BNP, 512), 0)
    return rows < NPB


def _stats_update(j, h, g, be, ac_out, s_sum, s_sq):
    @pl.when(j == 0)
    def _():
        s_sum[...] = jnp.zeros_like(s_sum)
        s_sq[...] = jnp.zeros_like(s_sq)

    h = jnp.where(_valid_mask(j), h, 0.0)
    s_sum[...] += jnp.sum(h, axis=0, keepdims=True)
    s_sq[...] += jnp.sum(h * h, axis=0, keepdims=True)

    @pl.when(j == NBP - 1)
    def _():
        add = lambda p, q: p + q
        mu = _fold8(s_sum[...], add) / N
        var = _fold8(s_sq[...], add) / N - mu * mu
        a = g[...] * lax.rsqrt(var + EPS)
        ac_out[0:1, :] = a
        ac_out[1:2, :] = be[...] - mu * a


def _l1_h(aggA, aggB, xpadP, Wl, Wr, b):
    agg = aggA[...] + aggB[...]
    dinvP = _dinv_packed(agg)
    meanp = agg * dinvP
    h = (jnp.dot(meanp, Wl[...], preferred_element_type=jnp.float32)
         + jnp.dot(xpadP[...], Wr[...], preferred_element_type=jnp.float32)
         + b[...])
    return h, dinvP


def _l1_stats_body(aggA, aggB, xpadP, Wl, Wr, b, g, be, ac_out, s_sum, s_sq):
    h, _ = _l1_h(aggA, aggB, xpadP, Wl, Wr, b)
    _stats_update(pl.program_id(0), h, g, be, ac_out, s_sum, s_sq)


def _l1_apply_body(aggA, aggB, xpadP, Wl, Wr, b, ac,
                   out0, out1, out2, out3, dinv_out):
    h, dinvP = _l1_h(aggA, aggB, xpadP, Wl, Wr, b)
    y = jnp.maximum(h * _tile8(ac[0:1, :]) + _tile8(ac[1:2, :]), 0.0)
    outs = (out0, out1, out2, out3)
    for t in range(NCHUNK):
        outs[t][...] = jnp.concatenate(
            [y[:, s * 64 + t * CW:s * 64 + t * CW + CW] for s in range(8)],
            axis=1)
    dinv_out[...] = dinvP


def _blkspecs():
    blk = pl.BlockSpec((BNP, 128), lambda j: (j, 0))
    blkB = pl.BlockSpec((BNP, 128), lambda j: (NBP + j, 0))
    wspec = lambda r, c: pl.BlockSpec((r, c), lambda j: (0, 0))
    return blk, blkB, wspec


def _tc_layer1(agg1P, xpadP, Wl, Wr, b, g, be):
    blk, blkB, wspec = _blkspecs()
    ac = pl.pallas_call(
        _l1_stats_body,
        grid=(NBP,),
        in_specs=[blk, blkB, blk, wspec(128, 512), wspec(128, 512),
                  wspec(1, 512), wspec(1, 64), wspec(1, 64)],
        out_specs=wspec(2, 64),
        out_shape=jax.ShapeDtypeStruct((2, 64), jnp.float32),
        scratch_shapes=[pltpu.VMEM((1, 512), jnp.float32),
                        pltpu.VMEM((1, 512), jnp.float32)],
    )(agg1P, agg1P, xpadP, Wl, Wr, b, g, be)
    outs = pl.pallas_call(
        _l1_apply_body,
        grid=(NBP,),
        in_specs=[blk, blkB, blk, wspec(128, 512), wspec(128, 512),
                  wspec(1, 512), wspec(2, 64)],
        out_specs=[blk] * 5,
        out_shape=[jax.ShapeDtypeStruct((NP, 128), jnp.float32)] * 5,
    )(agg1P, agg1P, xpadP, Wl, Wr, b, ac)
    return outs


def _l2_h(refs):
    (a0, a1, a2, a3,
     h0, h1c, h2c, h3, dinv_in, Wl, Wr, b) = refs
    agg = _to512([a0[...], a1[...], a2[...], a3[...]], CW)
    h1r = _to512([h0[...], h1c[...], h2c[...], h3[...]], CW)
    dinvP = dinv_in[...]
    dinv512 = jnp.concatenate(
        [jnp.broadcast_to(dinvP[:, s * 16:s * 16 + 1], (BNP, 64))
         for s in range(8)], axis=1)
    mean2 = agg * dinv512
    return (jnp.dot(mean2, Wl[...], preferred_element_type=jnp.float32)
            + jnp.dot(h1r, Wr[...], preferred_element_type=jnp.float32)
            + b[...])


def _l2_stats_body(*refs):
    ins, g, be, ac_out, s_sum, s_sq = (refs[:12], refs[12], refs[13],
                                       refs[14], refs[15], refs[16])
    h = _l2_h(ins)
    _stats_update(pl.program_id(0), h, g, be, ac_out, s_sum, s_sq)


def _l2_apply_body(*refs):
    ins, ac, out, s_max = refs[:12], refs[12], refs[13], refs[14]
    j = pl.program_id(0)
    h = _l2_h(ins)
    y = jnp.maximum(h * _tile8(ac[0:1, :]) + _tile8(ac[1:2, :]), 0.0)
    y = jnp.where(_valid_mask(j), y, 0.0)

    @pl.when(j == 0)
    def _():
        s_max[...] = jnp.zeros_like(s_max)

    s_max[...] = jnp.maximum(s_max[...], jnp.max(y, axis=0, keepdims=True))

    @pl.when(j == NBP - 1)
    def _():
        out[...] = _fold8(s_max[...], jnp.maximum)


def _tc_layer2(agg2Ps, h1cPs, dinvP, Wl, Wr, b, g, be):
    blk, blkB, wspec = _blkspecs()
    args = list(agg2Ps)
    specs = [blk] * 4
    args += list(h1cPs) + [dinvP, Wl, Wr, b]
    specs += [blk] * 5 + [wspec(512, 512), wspec(512, 512), wspec(1, 512)]
    ac = pl.pallas_call(
        _l2_stats_body,
        grid=(NBP,),
        in_specs=specs + [wspec(1, 64), wspec(1, 64)],
        out_specs=wspec(2, 64),
        out_shape=jax.ShapeDtypeStruct((2, 64), jnp.float32),
        scratch_shapes=[pltpu.VMEM((1, 512), jnp.float32),
                        pltpu.VMEM((1, 512), jnp.float32)],
    )(*args, g, be)
    return pl.pallas_call(
        _l2_apply_body,
        grid=(NBP,),
        in_specs=specs + [wspec(2, 64)],
        out_specs=wspec(1, 64),
        out_shape=jax.ShapeDtypeStruct((1, 64), jnp.float32),
        scratch_shapes=[pltpu.VMEM((1, 512), jnp.float32)],
    )(*args, ac)


def kernel(x, edge_index, W1l, b1l, W1r, g1, be1, W2l, b2l, W2r, g2, be2):
    f32 = jnp.float32
    xpad = jnp.concatenate(
        [x, jnp.ones((N, 1), f32), jnp.zeros((N, CW - 3), f32)], axis=1)
    xpad = jnp.pad(xpad, ((0, NPAD - N), (0, 0)))
    zero_blk = jnp.zeros((RPS, CW), f32)
    eye8 = jnp.eye(8, dtype=f32)

    W1l_p = jnp.zeros((CW, HID), f32).at[0:2].set(W1l)
    W1r_p = jnp.zeros((CW, HID), f32).at[0:2].set(W1r)
    W1l_big = jnp.kron(eye8, W1l_p)
    W1r_big = jnp.kron(eye8, W1r_p)
    W2l_big = jnp.kron(eye8, W2l)
    W2r_big = jnp.kron(eye8, W2r)
    b1t = jnp.tile(b1l.reshape(1, HID), (1, 8))
    b2t = jnp.tile(b2l.reshape(1, HID), (1, 8))

    agg1 = _sc_seg_sum_l1(edge_index, zero_blk, xpad)
    h1c0, h1c1, h1c2, h1c3, dinvP = _tc_layer1(
        agg1.reshape(2 * NP, 128), xpad.reshape(NP, 128),
        W1l_big, W1r_big, b1t, g1.reshape(1, HID), be1.reshape(1, HID))
    h1cPs = (h1c0, h1c1, h1c2, h1c3)
    agg2s = _sc_seg_sum_l2(edge_index, zero_blk,
                           [c.reshape(NPAD, CW) for c in h1cPs])
    out = _tc_layer2(
        [a.reshape(NP, 128) for a in agg2s], h1cPs, dinvP,
        W2l_big, W2r_big, b2t, g2.reshape(1, HID), be2.reshape(1, HID))
    return out.reshape(HID)
